# Initial kernel scaffold; baseline (speedup 1.0000x reference)
import jax
import jax.numpy as jnp
from jax.experimental import pallas as pl


def kernel(sample_items_embedd, all_items_embedd, random_slices):
    raise NotImplementedError("write your pallas kernel here")


# speedup vs baseline: 32.2468x; 32.2468x over previous
import functools

import jax
import jax.numpy as jnp
from jax import lax
from jax.experimental import pallas as pl
from jax.experimental.pallas import tpu as pltpu
from jax.experimental.pallas import tpu_sc as plsc

Q = 640
D = 64
N = 100000
NPAD = 100352
CB = 3584
GSTEPS = NPAD // CB
NBPC = CB // 128
NBLK = NPAD // 128
TOPK = 100
NES = 10
NW = 32


def _matmul_block(q_ref, a_ref):
    return jnp.dot(q_ref[...], a_ref[...].T, preferred_element_type=jnp.float32)


def _k0a_rowmax(q_ref, a_ref, out_ref, acc_ref):
    i = pl.program_id(0)

    @pl.when(i == 0)
    def _():
        acc_ref[...] = jnp.full((Q, 128), -jnp.inf, jnp.float32)

    s = _matmul_block(q_ref, a_ref)
    gcol = i * CB + lax.broadcasted_iota(jnp.int32, s.shape, 1)
    s = jnp.where(gcol < N, s, -jnp.inf)
    acc_ref[...] = jnp.maximum(acc_ref[...], jnp.max(s, axis=1, keepdims=True))

    @pl.when(i == GSTEPS - 1)
    def _():
        out_ref[...] = acc_ref[:, :1]


def _k0b_denom(q_ref, a_ref, rm_ref, s_out_ref, acc_ref):
    i = pl.program_id(0)

    @pl.when(i == 0)
    def _():
        acc_ref[...] = jnp.zeros((Q, 256), jnp.float32)

    s = _matmul_block(q_ref, a_ref)
    gcol = i * CB + lax.broadcasted_iota(jnp.int32, s.shape, 1)
    e = jnp.where(gcol < N, jnp.exp(s - rm_ref[...]), 0.0)
    a0 = acc_ref[:, :128]
    a1 = acc_ref[:, 128:]
    for c in range(NBPC):
        ec = e[:, c * 128:(c + 1) * 128]
        if c % 2 == 0:
            a0 = a0 + ec
        else:
            a1 = a1 + ec
    acc_ref[:, :128] = a0
    acc_ref[:, 128:] = a1

    @pl.when(i == GSTEPS - 1)
    def _():
        a2 = acc_ref[:, :128] + acc_ref[:, 128:]
        for k in (1, 2, 4, 8, 16, 32, 64):
            a2 = a2 + pltpu.roll(a2, 128 - k, 1)
        s_out_ref[...] = a2[:, :1]


def _k1b_softmax_bmax(q_ref, a_ref, rm_ref, den_ref, sm_ref, bm_ref):
    i = pl.program_id(0)
    s = _matmul_block(q_ref, a_ref)
    gcol = i * CB + lax.broadcasted_iota(jnp.int32, s.shape, 1)
    e = jnp.where(gcol < N, jnp.exp(s - rm_ref[...]), 0.0)
    smv = e / den_ref[...]
    sm_ref[...] = smv
    lane = lax.broadcasted_iota(jnp.int32, (Q, 128), 1)
    bm = jnp.full((Q, 128), -jnp.inf, jnp.float32)
    for c in range(NBPC):
        mc = jnp.max(smv[:, c * 128:(c + 1) * 128], axis=1, keepdims=True)
        bm = jnp.where(lane == c, mc, bm)
    bm_ref[...] = bm


def _k2_blocksel(bm_ref, selb_ref, v_ref):
    v_ref[...] = bm_ref[...]
    selb_ref[...] = jnp.zeros((Q, 128), jnp.int32)
    ii = lax.broadcasted_iota(jnp.int32, (Q, NBLK), 1)
    oi = lax.broadcasted_iota(jnp.int32, (Q, 128), 1)
    bigi = jnp.int32(2 ** 30)

    def step(t, carry):
        v = v_ref[...]
        m = jnp.max(v, axis=1, keepdims=True)
        cnd = jnp.where(v == m, ii, bigi)
        widx = jnp.min(cnd, axis=1, keepdims=True)
        v_ref[...] = jnp.where(ii == widx, -jnp.inf, v)
        selb_ref[...] = jnp.where(oi == t, widx, selb_ref[...])
        return carry

    lax.fori_loop(0, TOPK, step, 0)


QC = 80
NQC = Q // QC
CW = TOPK * 128


def _k4_extract(rs_ref, orig_ref, cand_ref, out_ref, vv_ref, sel_ref):
    vv_ref[...] = cand_ref[...]
    sel_ref[...] = jnp.zeros((QC, 128), jnp.int32)
    lane = lax.broadcasted_iota(jnp.int32, (QC, 128), 1)
    bigi = jnp.int32(2 ** 30)
    og = orig_ref[...]

    def stepfn(t, carry):
        v = vv_ref[...]
        m = jnp.max(v, axis=1, keepdims=True)
        c = jnp.where(v == m, og, bigi)
        widx = jnp.min(c, axis=1, keepdims=True)
        vv_ref[...] = jnp.where(og == widx, -jnp.inf, v)
        sel_ref[...] = jnp.where(lane == t, widx, sel_ref[...])
        return carry

    lax.fori_loop(0, TOPK, stepfn, 0)

    self32 = sel_ref[...].astype(jnp.float32)
    li = lax.broadcasted_iota(jnp.int32, (128, 16), 0)
    ti = lax.broadcasted_iota(jnp.int32, (128, 16), 1)
    oh = jnp.zeros((128, 16), jnp.float32)
    for t in range(NES):
        c = rs_ref[t]
        oh = jnp.where((ti == t) & (li == c), 1.0, oh)
    res = lax.dot(self32, oh, precision=lax.Precision.HIGHEST,
                  preferred_element_type=jnp.float32)
    out_ref[...] = res.astype(jnp.int32)


def _sc_gather(table, idx, rows, row_w, chunk):
    nchunks = rows // (NW * chunk)
    mesh = plsc.VectorSubcoreMesh(core_axis_name="c", subcore_axis_name="s")

    @functools.partial(
        pl.kernel,
        mesh=mesh,
        out_type=jax.ShapeDtypeStruct((rows, row_w), jnp.float32),
        scratch_types=[
            pltpu.VMEM((chunk,), jnp.int32),
            pltpu.VMEM((chunk, row_w), jnp.float32),
            pltpu.SemaphoreType.DMA,
        ],
    )
    def k(table_hbm, idx_hbm, out_hbm, idx_v, rows_v, sem):
        wid = lax.axis_index("s") * 2 + lax.axis_index("c")
        for ch in range(nchunks):
            base = wid * (rows // NW) + ch * chunk
            pltpu.sync_copy(idx_hbm.at[pl.ds(base, chunk)], idx_v)
            pltpu.async_copy(table_hbm.at[idx_v], rows_v, sem).wait()
            pltpu.sync_copy(rows_v, out_hbm.at[pl.ds(base, chunk)])

    return k(table, idx)


def kernel(sample_items_embedd, all_items_embedd, random_slices):
    qm = sample_items_embedd.reshape(Q, D)
    ap = jnp.pad(all_items_embedd, ((0, NPAD - N), (0, 0)))

    q_spec = pl.BlockSpec((Q, D), lambda i: (0, 0))
    a_spec = pl.BlockSpec((CB, D), lambda i: (i, 0))
    col_spec = pl.BlockSpec((Q, 1), lambda i: (0, 0))

    rowmax = pl.pallas_call(
        _k0a_rowmax,
        grid=(GSTEPS,),
        in_specs=[q_spec, a_spec],
        out_specs=pl.BlockSpec((Q, 1), lambda i: (0, 0)),
        out_shape=jax.ShapeDtypeStruct((Q, 1), jnp.float32),
        scratch_shapes=[pltpu.VMEM((Q, 128), jnp.float32)],
    )(qm, ap)

    denom = pl.pallas_call(
        _k0b_denom,
        grid=(GSTEPS,),
        in_specs=[q_spec, a_spec, col_spec],
        out_specs=pl.BlockSpec((Q, 1), lambda i: (0, 0)),
        out_shape=jax.ShapeDtypeStruct((Q, 1), jnp.float32),
        scratch_shapes=[pltpu.VMEM((Q, 256), jnp.float32)],
    )(qm, ap, rowmax)

    sm, bmax = pl.pallas_call(
        _k1b_softmax_bmax,
        grid=(GSTEPS,),
        in_specs=[q_spec, a_spec, col_spec, col_spec],
        out_specs=[
            pl.BlockSpec((Q, CB), lambda i: (0, i)),
            pl.BlockSpec((Q, 128), lambda i: (0, i)),
        ],
        out_shape=[
            jax.ShapeDtypeStruct((Q, NPAD), jnp.float32),
            jax.ShapeDtypeStruct((Q, GSTEPS * 128), jnp.float32),
        ],
    )(qm, ap, rowmax, denom)
    bmax = bmax.reshape(Q, GSTEPS, 128)[:, :, :NBPC].reshape(Q, NBLK)

    selb = pl.pallas_call(
        _k2_blocksel,
        in_specs=[pl.BlockSpec((Q, NBLK), lambda: (0, 0))],
        out_specs=pl.BlockSpec((Q, 128), lambda: (0, 0)),
        out_shape=jax.ShapeDtypeStruct((Q, 128), jnp.int32),
        scratch_shapes=[pltpu.VMEM((Q, NBLK), jnp.float32)],
    )(bmax)

    selb100 = selb[:, :TOPK]
    qrow = jnp.arange(Q, dtype=jnp.int32)[:, None]
    selg = (selb100 + NBLK * qrow).reshape(-1)

    cand = _sc_gather(sm.reshape(Q * NBLK, 128), selg, Q * TOPK, 128, 80)
    cand2 = cand.reshape(Q, CW)
    orig = (selb100[:, :, None] * 128 +
            jnp.arange(128, dtype=jnp.int32)).reshape(Q, CW)

    out10 = pl.pallas_call(
        _k4_extract,
        grid=(NQC,),
        in_specs=[
            pl.BlockSpec(memory_space=pltpu.SMEM),
            pl.BlockSpec((QC, CW), lambda i: (i, 0)),
            pl.BlockSpec((QC, CW), lambda i: (i, 0)),
        ],
        out_specs=pl.BlockSpec((QC, 16), lambda i: (i, 0)),
        out_shape=jax.ShapeDtypeStruct((Q, 16), jnp.int32),
        scratch_shapes=[
            pltpu.VMEM((QC, CW), jnp.float32),
            pltpu.VMEM((QC, 128), jnp.int32),
        ],
    )(random_slices, orig, cand2)

    idx_flat = out10[:, :NES].reshape(Q * NES)
    table128 = jnp.pad(all_items_embedd, ((0, 0), (0, 128 - D)))
    neg = _sc_gather(table128, idx_flat, Q * NES, 128, 40)
    return neg[:, :D].reshape(32, 20, NES, D)
